# baseline (speedup 1.0000x reference)
import functools

import jax
import jax.numpy as jnp
from jax import lax
from jax.experimental import pallas as pl
from jax.experimental.pallas import tpu as pltpu
from jax.experimental.pallas import tpu_sc as plsc

NC = 2
NS = 16
NW = NC * NS
LANES = 16
BATCH = 128
UNROLL = 8
COLS = 2048


def _tc_merge(val2d, rand2d, mask2d, sz):
  rows, cols = val2d.shape
  blk = 512

  def body(v_ref, r_ref, m_ref, sz_ref, o_ref):
    scale = sz_ref[0:1, 0:1]
    zp = sz_ref[1:2, 0:1]
    deq = (r_ref[...].astype(jnp.float32) - zp) * scale
    o_ref[...] = jnp.where(m_ref[...] != 0, deq, v_ref[...])

  return pl.pallas_call(
      body,
      out_shape=jax.ShapeDtypeStruct((rows, cols), val2d.dtype),
      grid=(rows // blk,),
      in_specs=[
          pl.BlockSpec((blk, cols), lambda i: (i, 0)),
          pl.BlockSpec((blk, cols), lambda i: (i, 0)),
          pl.BlockSpec((blk, cols), lambda i: (i, 0)),
          pl.BlockSpec((2, 1), lambda i: (0, 0)),
      ],
      out_specs=pl.BlockSpec((blk, cols), lambda i: (i, 0)),
  )(val2d, rand2d, mask2d, sz)


def _make_sc_mask_scatter(k_rows):
  mesh = plsc.VectorSubcoreMesh(core_axis_name="c", subcore_axis_name="s")

  @functools.partial(
      pl.kernel,
      mesh=mesh,
      scratch_types=[
          pltpu.VMEM((k_rows, BATCH), jnp.int32),
          pltpu.VMEM((BATCH,), jnp.int32),
          pltpu.SemaphoreType.DMA,
      ],
  )
  def sc_mask(mask_ref, idx_hbm, idx_v, ones_v, sem):
    wid = lax.axis_index("s") * NC + lax.axis_index("c")
    pltpu.sync_copy(idx_hbm.at[wid], idx_v)
    for k in range(BATCH // LANES):
      ones_v[pl.ds(k * LANES, LANES)] = jnp.ones((LANES,), jnp.int32)

    @pl.loop(0, k_rows // UNROLL)
    def _fire(r):
      for b in range(UNROLL):
        pltpu.async_copy(ones_v, mask_ref.at[idx_v.at[r * UNROLL + b]], sem)

    @pl.loop(0, k_rows // UNROLL)
    def _drain(r):
      for b in range(UNROLL):
        pltpu.make_async_copy(
            ones_v, mask_ref.at[idx_v.at[r * UNROLL + b]], sem).wait()

  return sc_mask


def kernel(value, scale, zero_point, flat_indices, rand_int):
  shape = value.shape
  total = value.size
  n_idx = flat_indices.shape[0]

  k_rows = -(-n_idx // (NW * BATCH))
  k_rows = -(-k_rows // UNROLL) * UNROLL
  n_pad = NW * k_rows * BATCH
  pad = jnp.broadcast_to(flat_indices[:1], (n_pad - n_idx,))
  idx3 = jnp.concatenate([flat_indices, pad]).reshape(NW, k_rows, BATCH)

  sz = jnp.stack([
      scale.astype(jnp.float32),
      zero_point.astype(jnp.float32),
  ])

  mask_ref = jax.new_ref(jnp.zeros((total,), jnp.int32))
  _make_sc_mask_scatter(k_rows)(mask_ref, idx3)
  mask2d = mask_ref[...].reshape(-1, COLS)

  out2d = _tc_merge(
      value.reshape(-1, COLS), rand_int.reshape(-1, COLS), mask2d, sz)
  return out2d.reshape(shape)


# speedup vs baseline: 2.8040x; 2.8040x over previous
import functools

import jax
import jax.numpy as jnp
from jax import lax
from jax.experimental import pallas as pl
from jax.experimental.pallas import tpu as pltpu
from jax.experimental.pallas import tpu_sc as plsc

NC = 2
NS = 16
NW = NC * NS
LANES = 16
BATCH = 128
COLS = 2048


def _tc_merge(val2d, rand2d, mask1d, sz):
  rows, cols = val2d.shape
  blk = 256

  def body(v_ref, r_ref, m_ref, sz_ref, o_ref):
    scale = sz_ref[0:1, 0:1]
    zp = sz_ref[1:2, 0:1]
    deq = (r_ref[...].astype(jnp.float32) - zp) * scale
    mask = m_ref[...].reshape(blk, cols)
    o_ref[...] = jnp.where(mask != 0, deq, v_ref[...])

  return pl.pallas_call(
      body,
      out_shape=jax.ShapeDtypeStruct((rows, cols), val2d.dtype),
      grid=(rows // blk,),
      in_specs=[
          pl.BlockSpec((blk, cols), lambda i: (i, 0)),
          pl.BlockSpec((blk, cols), lambda i: (i, 0)),
          pl.BlockSpec((blk * cols,), lambda i: (i,)),
          pl.BlockSpec((2, 1), lambda i: (0, 0)),
      ],
      out_specs=pl.BlockSpec((blk, cols), lambda i: (i, 0)),
  )(val2d, rand2d, mask1d, sz)


def _make_sc_mask_scatter(k_rows):
  mesh = plsc.VectorSubcoreMesh(core_axis_name="c", subcore_axis_name="s")

  @functools.partial(
      pl.kernel,
      mesh=mesh,
      scratch_types=[
          pltpu.VMEM((k_rows, BATCH), jnp.int32),
          pltpu.VMEM((BATCH,), jnp.int32),
          pltpu.SemaphoreType.DMA,
      ],
  )
  def sc_mask(mask_ref, idx_hbm, idx_v, ones_v, sem):
    wid = lax.axis_index("s") * NC + lax.axis_index("c")
    pltpu.sync_copy(idx_hbm.at[wid], idx_v)
    for k in range(BATCH // LANES):
      ones_v[pl.ds(k * LANES, LANES)] = jnp.ones((LANES,), jnp.int32)

    @pl.loop(0, k_rows)
    def _fire(r):
      pltpu.async_copy(ones_v, mask_ref.at[idx_v.at[r]], sem)

    @pl.loop(0, k_rows)
    def _drain(r):
      pltpu.make_async_copy(ones_v, mask_ref.at[idx_v.at[r]], sem).wait()

  return sc_mask


def kernel(value, scale, zero_point, flat_indices, rand_int):
  shape = value.shape
  total = value.size
  n_idx = flat_indices.shape[0]

  k_rows = -(-n_idx // (NW * BATCH))
  n_pad = NW * k_rows * BATCH
  pad = jnp.broadcast_to(flat_indices[:1], (n_pad - n_idx,))
  idx3 = jnp.concatenate([flat_indices, pad]).reshape(NW, k_rows, BATCH)

  sz = jnp.stack([
      scale.astype(jnp.float32),
      zero_point.astype(jnp.float32),
  ])

  mask_ref = jax.new_ref(jnp.zeros((total,), jnp.int32))
  _make_sc_mask_scatter(k_rows)(mask_ref, idx3)

  out2d = _tc_merge(
      value.reshape(-1, COLS), rand_int.reshape(-1, COLS), mask_ref[...], sz)
  return out2d.reshape(shape)


# speedup vs baseline: 2.8061x; 1.0008x over previous
import functools

import jax
import jax.numpy as jnp
from jax import lax
from jax.experimental import pallas as pl
from jax.experimental.pallas import tpu as pltpu
from jax.experimental.pallas import tpu_sc as plsc

NC = 2
NS = 16
NW = NC * NS
LANES = 16
BATCH = 128
COLS = 2048


def _tc_merge(val2d, rand2d, mask1d, sz):
  rows, cols = val2d.shape
  blk = 256

  def body(v_ref, r_ref, m_ref, sz_ref, o_ref):
    scale = sz_ref[0:1, 0:1]
    zp = sz_ref[1:2, 0:1]
    deq = (r_ref[...].astype(jnp.float32) - zp) * scale
    mask = m_ref[...].reshape(blk, cols)
    o_ref[...] = jnp.where(mask != 0, deq, v_ref[...])

  return pl.pallas_call(
      body,
      out_shape=jax.ShapeDtypeStruct((rows, cols), val2d.dtype),
      grid=(rows // blk,),
      in_specs=[
          pl.BlockSpec((blk, cols), lambda i: (i, 0)),
          pl.BlockSpec((blk, cols), lambda i: (i, 0)),
          pl.BlockSpec((blk * cols,), lambda i: (i,)),
          pl.BlockSpec((2, 1), lambda i: (0, 0)),
      ],
      out_specs=pl.BlockSpec((blk, cols), lambda i: (i, 0)),
  )(val2d, rand2d, mask1d, sz)


def _tc_zero(total):
  blk = 1024 * 1024

  return pl.pallas_call(
      lambda o_ref: o_ref.__setitem__((...,), jnp.zeros((blk,), jnp.int32)),
      out_shape=jax.ShapeDtypeStruct((total,), jnp.int32),
      grid=(total // blk,),
      out_specs=pl.BlockSpec((blk,), lambda i: (i,)),
  )()


def _make_sc_mask_scatter(k_rows):
  mesh = plsc.VectorSubcoreMesh(core_axis_name="c", subcore_axis_name="s")

  @functools.partial(
      pl.kernel,
      mesh=mesh,
      scratch_types=[
          pltpu.VMEM((k_rows, BATCH), jnp.int32),
          pltpu.VMEM((BATCH,), jnp.int32),
          pltpu.SemaphoreType.DMA,
      ],
  )
  def sc_mask(mask_ref, idx_hbm, idx_v, ones_v, sem):
    wid = lax.axis_index("s") * NC + lax.axis_index("c")
    pltpu.sync_copy(idx_hbm.at[wid], idx_v)
    for k in range(BATCH // LANES):
      ones_v[pl.ds(k * LANES, LANES)] = jnp.ones((LANES,), jnp.int32)

    @pl.loop(0, k_rows)
    def _fire(r):
      pltpu.async_copy(ones_v, mask_ref.at[idx_v.at[r]], sem)

    @pl.loop(0, k_rows)
    def _drain(r):
      pltpu.make_async_copy(ones_v, mask_ref.at[idx_v.at[r]], sem).wait()

  return sc_mask


def kernel(value, scale, zero_point, flat_indices, rand_int):
  shape = value.shape
  total = value.size
  n_idx = flat_indices.shape[0]

  k_rows = -(-n_idx // (NW * BATCH))
  n_pad = NW * k_rows * BATCH
  pad = jnp.broadcast_to(flat_indices[:1], (n_pad - n_idx,))
  idx3 = jnp.concatenate([flat_indices, pad]).reshape(NW, k_rows, BATCH)

  sz = jnp.stack([
      scale.astype(jnp.float32),
      zero_point.astype(jnp.float32),
  ])

  mask_ref = jax.new_ref(_tc_zero(total))
  _make_sc_mask_scatter(k_rows)(mask_ref, idx3)

  out2d = _tc_merge(
      value.reshape(-1, COLS), rand_int.reshape(-1, COLS), mask_ref[...], sz)
  return out2d.reshape(shape)
